# baseline (speedup 1.0000x reference)
import functools

import jax
import jax.numpy as jnp
import numpy as np
from jax import lax
from jax.experimental import pallas as pl
from jax.experimental.pallas import tpu as pltpu
from jax.experimental.pallas import tpu_sc as plsc

N = 10000
E = 320000
D = 128
H = 256
C = 10
G = 64
NC = 2
NS = 16
NP = 10240
HW = 128
CHUNK = 128
NCH = 158
EP = NS * NCH * CHUNK
NCH0 = 80
EP0 = NS * NCH0 * CHUNK
RPT = NP // NS
RB = 512
NB = NP // RB


def _make_agg(nch):
    mesh = plsc.VectorSubcoreMesh(core_axis_name="c", subcore_axis_name="s")

    @functools.partial(
        pl.kernel,
        out_type=jax.ShapeDtypeStruct((NC, NP, HW), jnp.float32),
        mesh=mesh,
        scratch_types=[
            pltpu.VMEM((2, CHUNK), jnp.int32),
            pltpu.VMEM((2, CHUNK), jnp.int32),
            pltpu.VMEM((CHUNK, HW), jnp.float32),
            pltpu.VMEM_SHARED((NP, HW), jnp.float32),
            pltpu.SemaphoreType.DMA,
        ],
    )
    def agg_kernel(table, idx_st, zeros_hbm, out, ia, ib, rows, acc, sem):
        cid = lax.axis_index("c")
        tid = lax.axis_index("s")
        pltpu.sync_copy(zeros_hbm, acc.at[pl.ds(tid * RPT, RPT)])
        plsc.subcore_barrier()
        pltpu.sync_copy(idx_st.at[cid, tid, 0], ia)

        def body(p, carry):
            cp = pltpu.async_copy(table.at[ia.at[0]], rows, sem)
            pltpu.sync_copy(idx_st.at[cid, tid, 2 * p + 1], ib)
            cp.wait()
            pltpu.sync_copy(rows, acc.at[ia.at[1]], add=True)
            cp = pltpu.async_copy(table.at[ib.at[0]], rows, sem)
            pltpu.sync_copy(idx_st.at[cid, tid, 2 * p + 2], ia)
            cp.wait()
            pltpu.sync_copy(rows, acc.at[ib.at[1]], add=True)
            return carry

        lax.fori_loop(0, nch // 2, body, 0)
        plsc.subcore_barrier()
        pltpu.sync_copy(acc.at[pl.ds(tid * RPT, RPT)],
                        out.at[cid, pl.ds(tid * RPT, RPT)])

    return agg_kernel


def _mlp_body(agg_ref, x_ref, w1_ref, b1_ref, w2_ref, b2_ref, gs_ref, bt_ref,
              c_ref, o_ref):
    b = pl.program_id(0)
    cm = c_ref[0, 0]
    a0 = agg_ref[0] + cm * x_ref[0]
    a1 = agg_ref[1] + cm * x_ref[1]
    h = jnp.dot(a0, w1_ref[0], preferred_element_type=jnp.float32)
    h = h + jnp.dot(a1, w1_ref[1], preferred_element_type=jnp.float32)
    h = jnp.maximum(h + b1_ref[...], 0.0)
    y = jnp.dot(h, w2_ref[...], preferred_element_type=jnp.float32)
    y = jnp.maximum(y + b2_ref[...], 0.0)
    y = y * gs_ref[...] + bt_ref[...]
    rows = b * RB + lax.broadcasted_iota(jnp.int32, (RB, 1), 0)
    y = jnp.where(rows < N, y, 0.0)
    o_ref[0] = y[:, : H // 2]
    o_ref[1] = y[:, H // 2:]


def _mlp_call(agg, xs, w1s, b1, w2, b2, gs, bt, cm):
    return pl.pallas_call(
        _mlp_body,
        grid=(NB,),
        in_specs=[
            pl.BlockSpec((2, RB, HW), lambda b: (0, b, 0)),
            pl.BlockSpec((2, RB, HW), lambda b: (0, b, 0)),
            pl.BlockSpec((2, HW, H), lambda b: (0, 0, 0)),
            pl.BlockSpec((1, H), lambda b: (0, 0)),
            pl.BlockSpec((H, H), lambda b: (0, 0)),
            pl.BlockSpec((1, H), lambda b: (0, 0)),
            pl.BlockSpec((1, H), lambda b: (0, 0)),
            pl.BlockSpec((1, H), lambda b: (0, 0)),
            pl.BlockSpec((1, 1), lambda b: (0, 0)),
        ],
        out_specs=pl.BlockSpec((2, RB, H // 2), lambda b: (0, b, 0)),
        out_shape=jax.ShapeDtypeStruct((2, NP, H // 2), jnp.float32),
    )(agg, xs, w1s, b1, w2, b2, gs, bt, cm)


def _head_body(x_ref, b_ref, w1_ref, b1_ref, w2_ref, b2_ref, o_ref,
               s0, s1, cnt):
    b = pl.program_id(0)

    @pl.when(b == 0)
    def _init():
        s0[...] = jnp.zeros((G, H // 2), jnp.float32)
        s1[...] = jnp.zeros((G, H // 2), jnp.float32)
        cnt[...] = jnp.zeros((G, 1), jnp.float32)

    bid = b_ref[0]
    iog = lax.broadcasted_iota(jnp.int32, (G, RB), 0)
    oh = (iog == jnp.broadcast_to(bid, (G, RB))).astype(jnp.float32)
    s0[...] += jnp.dot(oh, x_ref[0], preferred_element_type=jnp.float32)
    s1[...] += jnp.dot(oh, x_ref[1], preferred_element_type=jnp.float32)
    cnt[...] += jnp.sum(oh, axis=1, keepdims=True)

    @pl.when(b == NB - 1)
    def _final():
        cc = jnp.maximum(cnt[...], 1.0)
        p = jnp.concatenate([s0[...] / cc, s1[...] / cc], axis=1)
        h = jnp.dot(p, w1_ref[...], preferred_element_type=jnp.float32)
        h = jnp.maximum(h + b1_ref[...], 0.0)
        lg = jnp.dot(h, w2_ref[...], preferred_element_type=jnp.float32)
        lg = lg + b2_ref[...]
        m = jnp.max(lg, axis=1, keepdims=True)
        lse = jnp.log(jnp.sum(jnp.exp(lg - m), axis=1, keepdims=True)) + m
        o_ref[...] = lg - lse


def _head_call(xs, batch3, l1w, l1b, l2w, l2b):
    return pl.pallas_call(
        _head_body,
        grid=(NB,),
        in_specs=[
            pl.BlockSpec((2, RB, H // 2), lambda b: (0, b, 0)),
            pl.BlockSpec((1, 1, RB), lambda b: (b, 0, 0)),
            pl.BlockSpec((H, H), lambda b: (0, 0)),
            pl.BlockSpec((1, H), lambda b: (0, 0)),
            pl.BlockSpec((H, 128), lambda b: (0, 0)),
            pl.BlockSpec((1, 128), lambda b: (0, 0)),
        ],
        out_specs=pl.BlockSpec((G, 128), lambda b: (0, 0)),
        out_shape=jax.ShapeDtypeStruct((G, 128), jnp.float32),
        scratch_shapes=[
            pltpu.VMEM((G, H // 2), jnp.float32),
            pltpu.VMEM((G, H // 2), jnp.float32),
            pltpu.VMEM((G, 1), jnp.float32),
        ],
    )(xs, batch3, l1w, l1b, l2w, l2b)


def kernel(x, edge_index, batch,
           conv0_eps, conv0_W1, conv0_b1, conv0_W2, conv0_b2, conv0_g, conv0_bt,
           conv1_eps, conv1_W1, conv1_b1, conv1_W2, conv1_b2, conv1_g, conv1_bt,
           conv2_eps, conv2_W1, conv2_b1, conv2_W2, conv2_b2, conv2_g, conv2_bt,
           lin1_W, lin1_b, lin2_W, lin2_b):
    f32 = jnp.float32
    src = edge_index[0]
    dst = edge_index[1]

    he = E // 2
    pad0 = jnp.full((EP0 - he,), N, jnp.int32)
    src0 = jnp.stack([jnp.concatenate([src[:he], pad0]),
                      jnp.concatenate([src[he:], pad0])]).reshape(
                          NC, NS, NCH0, CHUNK)
    dst0 = jnp.stack([jnp.concatenate([dst[:he], pad0]),
                      jnp.concatenate([dst[he:], pad0])]).reshape(
                          NC, NS, NCH0, CHUNK)
    idx0 = jnp.pad(jnp.stack([src0, dst0], axis=3),
                   ((0, 0), (0, 0), (0, 1), (0, 0), (0, 0)),
                   constant_values=N)

    padh = jnp.full((EP - E,), N, jnp.int32)
    src_p = jnp.concatenate([src, padh])
    dst_p = jnp.concatenate([dst, padh])
    srch = jnp.stack([src_p, src_p + NP]).reshape(NC, NS, NCH, CHUNK)
    dsth = jnp.stack([dst_p, dst_p]).reshape(NC, NS, NCH, CHUNK)
    idxh = jnp.pad(jnp.stack([srch, dsth], axis=3),
                   ((0, 0), (0, 0), (0, 1), (0, 0), (0, 0)),
                   constant_values=N)

    batch3 = jnp.concatenate(
        [batch, jnp.full((NP - N,), -1, jnp.int32)]).reshape(NB, 1, RB)

    xp = jnp.pad(x, ((0, NP - N), (0, 0)))
    zeros_h = jnp.zeros((RPT, HW), f32)
    agg_e = _make_agg(NCH0)
    agg_h = _make_agg(NCH)
    bninv = f32(1.0 / np.sqrt(1.0 + 1e-5))

    convs = [
        (conv0_eps, conv0_W1, conv0_b1, conv0_W2, conv0_b2, conv0_g, conv0_bt),
        (conv1_eps, conv1_W1, conv1_b1, conv1_W2, conv1_b2, conv1_g, conv1_bt),
        (conv2_eps, conv2_W1, conv2_b1, conv2_W2, conv2_b2, conv2_g, conv2_bt),
    ]
    xs = None
    for i, (eps, w1, b1, w2, b2, g, bt) in enumerate(convs):
        if i == 0:
            agg = agg_e(xp, idx0, zeros_h)
            xin = jnp.stack([xp, jnp.zeros_like(xp)])
            w1s = jnp.stack([w1, w1])
        else:
            table = xs.reshape(NC * NP, HW)
            agg = agg_h(table, idxh, zeros_h)
            xin = xs
            w1s = jnp.stack([w1[:HW], w1[HW:]])
        cm = (1.0 + eps[0]).reshape(1, 1).astype(f32)
        xs = _mlp_call(agg, xin, w1s, b1.reshape(1, H), w2, b2.reshape(1, H),
                       (g * bninv).reshape(1, H), bt.reshape(1, H), cm)

    l2w_p = jnp.pad(lin2_W, ((0, 0), (0, 128 - C)))
    l2b_p = jnp.pad(lin2_b, (0, 128 - C), constant_values=-1e30).reshape(1, 128)
    out = _head_call(xs, batch3, lin1_W, lin1_b.reshape(1, H), l2w_p, l2b_p)
    return out[:, :C]


# speedup vs baseline: 1.0907x; 1.0907x over previous
import functools

import jax
import jax.numpy as jnp
import numpy as np
from jax import lax
from jax.experimental import pallas as pl
from jax.experimental.pallas import tpu as pltpu
from jax.experimental.pallas import tpu_sc as plsc

N = 10000
E = 320000
D = 128
H = 256
C = 10
G = 64
NC = 2
NS = 16
NP = 10240
HW = 128
CHUNK = 128
NCH = 157
EP = NS * NCH * CHUNK
NCH0 = 79
EP0 = NS * NCH0 * CHUNK
RPT = NP // NS
RB = 512
NB = NP // RB


def _make_agg(nch):
    mesh = plsc.VectorSubcoreMesh(core_axis_name="c", subcore_axis_name="s")

    @functools.partial(
        pl.kernel,
        out_type=jax.ShapeDtypeStruct((NC, NP, HW), jnp.float32),
        mesh=mesh,
        scratch_types=[
            pltpu.VMEM((2, CHUNK), jnp.int32),
            pltpu.VMEM((CHUNK, HW), jnp.float32),
            pltpu.VMEM_SHARED((NP, HW), jnp.float32),
            pltpu.SemaphoreType.DMA,
        ],
    )
    def agg_kernel(table, idx_st, zeros_hbm, out, idx2, rows, acc, sem):
        cid = lax.axis_index("c")
        tid = lax.axis_index("s")
        pltpu.sync_copy(zeros_hbm, acc.at[pl.ds(tid * RPT, RPT)])
        plsc.subcore_barrier()

        def body(j, carry):
            pltpu.sync_copy(idx_st.at[cid, tid, j], idx2)
            pltpu.async_copy(table.at[idx2.at[0]], rows, sem).wait()
            pltpu.sync_copy(rows, acc.at[idx2.at[1]], add=True)
            return carry

        lax.fori_loop(0, nch, body, 0)
        plsc.subcore_barrier()
        pltpu.sync_copy(acc.at[pl.ds(tid * RPT, RPT)],
                        out.at[cid, pl.ds(tid * RPT, RPT)])

    return agg_kernel


def _mlp_body(agg_ref, x_ref, w1_ref, b1_ref, w2_ref, b2_ref, gs_ref, bt_ref,
              c_ref, o_ref):
    b = pl.program_id(0)
    cm = c_ref[0, 0]
    a0 = agg_ref[0] + cm * x_ref[0]
    a1 = agg_ref[1] + cm * x_ref[1]
    h = jnp.dot(a0, w1_ref[0], preferred_element_type=jnp.float32)
    h = h + jnp.dot(a1, w1_ref[1], preferred_element_type=jnp.float32)
    h = jnp.maximum(h + b1_ref[...], 0.0)
    y = jnp.dot(h, w2_ref[...], preferred_element_type=jnp.float32)
    y = jnp.maximum(y + b2_ref[...], 0.0)
    y = y * gs_ref[...] + bt_ref[...]
    rows = b * RB + lax.broadcasted_iota(jnp.int32, (RB, 1), 0)
    y = jnp.where(rows < N, y, 0.0)
    o_ref[0] = y[:, : H // 2]
    o_ref[1] = y[:, H // 2:]


def _mlp_call(agg, xs, w1s, b1, w2, b2, gs, bt, cm):
    return pl.pallas_call(
        _mlp_body,
        grid=(NB,),
        in_specs=[
            pl.BlockSpec((2, RB, HW), lambda b: (0, b, 0)),
            pl.BlockSpec((2, RB, HW), lambda b: (0, b, 0)),
            pl.BlockSpec((2, HW, H), lambda b: (0, 0, 0)),
            pl.BlockSpec((1, H), lambda b: (0, 0)),
            pl.BlockSpec((H, H), lambda b: (0, 0)),
            pl.BlockSpec((1, H), lambda b: (0, 0)),
            pl.BlockSpec((1, H), lambda b: (0, 0)),
            pl.BlockSpec((1, H), lambda b: (0, 0)),
            pl.BlockSpec((1, 1), lambda b: (0, 0)),
        ],
        out_specs=pl.BlockSpec((2, RB, H // 2), lambda b: (0, b, 0)),
        out_shape=jax.ShapeDtypeStruct((2, NP, H // 2), jnp.float32),
    )(agg, xs, w1s, b1, w2, b2, gs, bt, cm)


def _head_body(x_ref, b_ref, w1_ref, b1_ref, w2_ref, b2_ref, o_ref,
               s0, s1, cnt):
    b = pl.program_id(0)

    @pl.when(b == 0)
    def _init():
        s0[...] = jnp.zeros((G, H // 2), jnp.float32)
        s1[...] = jnp.zeros((G, H // 2), jnp.float32)
        cnt[...] = jnp.zeros((G, 1), jnp.float32)

    bid = b_ref[0]
    iog = lax.broadcasted_iota(jnp.int32, (G, RB), 0)
    oh = (iog == jnp.broadcast_to(bid, (G, RB))).astype(jnp.float32)
    s0[...] += jnp.dot(oh, x_ref[0], preferred_element_type=jnp.float32)
    s1[...] += jnp.dot(oh, x_ref[1], preferred_element_type=jnp.float32)
    cnt[...] += jnp.sum(oh, axis=1, keepdims=True)

    @pl.when(b == NB - 1)
    def _final():
        cc = jnp.maximum(cnt[...], 1.0)
        p = jnp.concatenate([s0[...] / cc, s1[...] / cc], axis=1)
        h = jnp.dot(p, w1_ref[...], preferred_element_type=jnp.float32)
        h = jnp.maximum(h + b1_ref[...], 0.0)
        lg = jnp.dot(h, w2_ref[...], preferred_element_type=jnp.float32)
        lg = lg + b2_ref[...]
        m = jnp.max(lg, axis=1, keepdims=True)
        lse = jnp.log(jnp.sum(jnp.exp(lg - m), axis=1, keepdims=True)) + m
        o_ref[...] = lg - lse


def _head_call(xs, batch3, l1w, l1b, l2w, l2b):
    return pl.pallas_call(
        _head_body,
        grid=(NB,),
        in_specs=[
            pl.BlockSpec((2, RB, H // 2), lambda b: (0, b, 0)),
            pl.BlockSpec((1, 1, RB), lambda b: (b, 0, 0)),
            pl.BlockSpec((H, H), lambda b: (0, 0)),
            pl.BlockSpec((1, H), lambda b: (0, 0)),
            pl.BlockSpec((H, 128), lambda b: (0, 0)),
            pl.BlockSpec((1, 128), lambda b: (0, 0)),
        ],
        out_specs=pl.BlockSpec((G, 128), lambda b: (0, 0)),
        out_shape=jax.ShapeDtypeStruct((G, 128), jnp.float32),
        scratch_shapes=[
            pltpu.VMEM((G, H // 2), jnp.float32),
            pltpu.VMEM((G, H // 2), jnp.float32),
            pltpu.VMEM((G, 1), jnp.float32),
        ],
    )(xs, batch3, l1w, l1b, l2w, l2b)


def kernel(x, edge_index, batch,
           conv0_eps, conv0_W1, conv0_b1, conv0_W2, conv0_b2, conv0_g, conv0_bt,
           conv1_eps, conv1_W1, conv1_b1, conv1_W2, conv1_b2, conv1_g, conv1_bt,
           conv2_eps, conv2_W1, conv2_b1, conv2_W2, conv2_b2, conv2_g, conv2_bt,
           lin1_W, lin1_b, lin2_W, lin2_b):
    f32 = jnp.float32
    src = edge_index[0]
    dst = edge_index[1]

    he = E // 2
    pad0 = jnp.full((EP0 - he,), N, jnp.int32)
    src0 = jnp.stack([jnp.concatenate([src[:he], pad0]),
                      jnp.concatenate([src[he:], pad0])]).reshape(
                          NC, NS, NCH0, CHUNK)
    dst0 = jnp.stack([jnp.concatenate([dst[:he], pad0]),
                      jnp.concatenate([dst[he:], pad0])]).reshape(
                          NC, NS, NCH0, CHUNK)
    idx0 = jnp.stack([src0, dst0], axis=3)

    padh = jnp.full((EP - E,), N, jnp.int32)
    src_p = jnp.concatenate([src, padh])
    dst_p = jnp.concatenate([dst, padh])
    srch = jnp.stack([src_p, src_p + NP]).reshape(NC, NS, NCH, CHUNK)
    dsth = jnp.stack([dst_p, dst_p]).reshape(NC, NS, NCH, CHUNK)
    idxh = jnp.stack([srch, dsth], axis=3)

    batch3 = jnp.concatenate(
        [batch, jnp.full((NP - N,), -1, jnp.int32)]).reshape(NB, 1, RB)

    xp = jnp.pad(x, ((0, NP - N), (0, 0)))
    zeros_h = jnp.zeros((RPT, HW), f32)
    agg_e = _make_agg(NCH0)
    agg_h = _make_agg(NCH)
    bninv = f32(1.0 / np.sqrt(1.0 + 1e-5))

    convs = [
        (conv0_eps, conv0_W1, conv0_b1, conv0_W2, conv0_b2, conv0_g, conv0_bt),
        (conv1_eps, conv1_W1, conv1_b1, conv1_W2, conv1_b2, conv1_g, conv1_bt),
        (conv2_eps, conv2_W1, conv2_b1, conv2_W2, conv2_b2, conv2_g, conv2_bt),
    ]
    xs = None
    for i, (eps, w1, b1, w2, b2, g, bt) in enumerate(convs):
        if i == 0:
            agg = agg_e(xp, idx0, zeros_h)
            xin = jnp.stack([xp, jnp.zeros_like(xp)])
            w1s = jnp.stack([w1, w1])
        else:
            table = xs.reshape(NC * NP, HW)
            agg = agg_h(table, idxh, zeros_h)
            xin = xs
            w1s = jnp.stack([w1[:HW], w1[HW:]])
        cm = (1.0 + eps[0]).reshape(1, 1).astype(f32)
        xs = _mlp_call(agg, xin, w1s, b1.reshape(1, H), w2, b2.reshape(1, H),
                       (g * bninv).reshape(1, H), bt.reshape(1, H), cm)

    l2w_p = jnp.pad(lin2_W, ((0, 0), (0, 128 - C)))
    l2b_p = jnp.pad(lin2_b, (0, 128 - C), constant_values=-1e30).reshape(1, 128)
    out = _head_call(xs, batch3, lin1_W, lin1_b.reshape(1, H), l2w_p, l2b_p)
    return out[:, :C]
